# baseline (speedup 1.0000x reference)
import functools

import jax
import jax.numpy as jnp
from jax import lax
from jax.experimental import pallas as pl
from jax.experimental.pallas import tpu as pltpu
from jax.experimental.pallas import tpu_sc as plsc

BATCH = 16384
DIM = 64
NC = 2
NS = 16
L = 16
NW = NC * NS
B_PER_W = BATCH // NW
CHUNK = 128
NCHUNK = B_PER_W // CHUNK
GROUPS = CHUNK // L


def _make_sc_kernel():
    mesh = plsc.VectorSubcoreMesh(core_axis_name="c", subcore_axis_name="s")

    @functools.partial(
        pl.kernel,
        out_type=jax.ShapeDtypeStruct((NW, L), jnp.float32),
        mesh=mesh,
        compiler_params=pltpu.CompilerParams(
            needs_layout_passes=False, use_tc_tiling_on_sc=False),
        scratch_types=[
            pltpu.VMEM((B_PER_W,), jnp.int32),
            pltpu.VMEM((B_PER_W,), jnp.int32),
            pltpu.VMEM((B_PER_W,), jnp.int32),
            pltpu.VMEM((B_PER_W,), jnp.int32),
            pltpu.VMEM((CHUNK, DIM), jnp.float32),
            pltpu.VMEM((CHUNK, DIM), jnp.float32),
            pltpu.VMEM((CHUNK, DIM), jnp.float32),
            pltpu.VMEM((CHUNK, DIM), jnp.float32),
            pltpu.VMEM((L,), jnp.float32),
            pltpu.SemaphoreType.DMA,
        ],
    )
    def sc_kernel(h_hbm, r_hbm, t_hbm, n_hbm, ent_hbm, rel_hbm, out_hbm,
                  h_v, r_v, t_v, n_v, he_v, re_v, te_v, ne_v, o_v, sem):
        wid = lax.axis_index("s") * NC + lax.axis_index("c")
        base = wid * B_PER_W
        pltpu.sync_copy(h_hbm.at[pl.ds(base, B_PER_W)], h_v)
        pltpu.sync_copy(r_hbm.at[pl.ds(base, B_PER_W)], r_v)
        pltpu.sync_copy(t_hbm.at[pl.ds(base, B_PER_W)], t_v)
        pltpu.sync_copy(n_hbm.at[pl.ds(base, B_PER_W)], n_v)

        lane = lax.iota(jnp.int32, L)
        total = jnp.zeros((L,), jnp.float32)
        for c in range(NCHUNK):
            off = c * CHUNK
            d1 = pltpu.async_copy(ent_hbm.at[h_v.at[pl.ds(off, CHUNK)]], he_v, sem)
            d2 = pltpu.async_copy(rel_hbm.at[r_v.at[pl.ds(off, CHUNK)]], re_v, sem)
            d3 = pltpu.async_copy(ent_hbm.at[t_v.at[pl.ds(off, CHUNK)]], te_v, sem)
            d4 = pltpu.async_copy(ent_hbm.at[n_v.at[pl.ds(off, CHUNK)]], ne_v, sem)
            d1.wait()
            d2.wait()
            d3.wait()
            d4.wait()

            def group_body(g, tot):
                rows = g * L + lane

                def col_body(col, acc):
                    cols = jnp.full((L,), col, jnp.int32)
                    sh = plsc.load_gather(he_v, [rows, cols])
                    sr = plsc.load_gather(re_v, [rows, cols])
                    st = plsc.load_gather(te_v, [rows, cols])
                    sn = plsc.load_gather(ne_v, [rows, cols])
                    s = sh + sr
                    return acc + (jnp.abs(s - st) - jnp.abs(s - sn))

                acc = lax.fori_loop(0, DIM, col_body, jnp.zeros((L,), jnp.float32))
                return tot + jnp.maximum(acc + 1.0, 0.0)

            total = lax.fori_loop(0, GROUPS, group_body, total)

        o_v[...] = total
        pltpu.sync_copy(o_v, out_hbm.at[wid])

    return sc_kernel


_sc_kernel = _make_sc_kernel()


def _tc_reduce(partials):
    def body(x_ref, o_ref):
        o_ref[...] = jnp.sum(x_ref[...], keepdims=True).reshape(1, 1) * (1.0 / BATCH)

    return pl.pallas_call(
        body,
        out_shape=jax.ShapeDtypeStruct((1, 1), jnp.float32),
    )(partials)


def kernel(h, r, t, neg_t, entity_emb, relation_emb):
    h = h.astype(jnp.int32)
    r = r.astype(jnp.int32)
    t = t.astype(jnp.int32)
    neg_t = neg_t.astype(jnp.int32)
    partials = _sc_kernel(h, r, t, neg_t, entity_emb, relation_emb)
    return _tc_reduce(partials).reshape(())


# speedup vs baseline: 1.0919x; 1.0919x over previous
import functools

import jax
import jax.numpy as jnp
from jax import lax
from jax.experimental import pallas as pl
from jax.experimental.pallas import tpu as pltpu
from jax.experimental.pallas import tpu_sc as plsc

BATCH = 16384
DIM = 64
NC = 2
NS = 16
L = 16
NW = NC * NS
B_PER_W = BATCH // NW
CHUNK = 128
NCHUNK = B_PER_W // CHUNK
GROUPS = CHUNK // L


def _make_sc_kernel():
    mesh = plsc.VectorSubcoreMesh(core_axis_name="c", subcore_axis_name="s")

    @functools.partial(
        pl.kernel,
        out_type=jax.ShapeDtypeStruct((NW * L,), jnp.float32),
        mesh=mesh,
        compiler_params=pltpu.CompilerParams(needs_layout_passes=False),
        scratch_types=[
            pltpu.VMEM((B_PER_W,), jnp.int32),
            pltpu.VMEM((B_PER_W,), jnp.int32),
            pltpu.VMEM((B_PER_W,), jnp.int32),
            pltpu.VMEM((B_PER_W,), jnp.int32),
            pltpu.VMEM((CHUNK, 2 * DIM), jnp.float32),
            pltpu.VMEM((CHUNK, 2 * DIM), jnp.float32),
            pltpu.VMEM((CHUNK, 2 * DIM), jnp.float32),
            pltpu.VMEM((CHUNK, 2 * DIM), jnp.float32),
            pltpu.VMEM((L,), jnp.float32),
            pltpu.SemaphoreType.DMA,
        ],
    )
    def sc_kernel(h_hbm, r_hbm, t_hbm, n_hbm, ent_hbm, rel_hbm, out_hbm,
                  h_v, r_v, t_v, n_v, he_v, re_v, te_v, ne_v, o_v, sem):
        wid = lax.axis_index("s") * NC + lax.axis_index("c")
        base = wid * B_PER_W
        pltpu.sync_copy(h_hbm.at[pl.ds(base, B_PER_W)], h_v)
        pltpu.sync_copy(r_hbm.at[pl.ds(base, B_PER_W)], r_v)
        pltpu.sync_copy(t_hbm.at[pl.ds(base, B_PER_W)], t_v)
        pltpu.sync_copy(n_hbm.at[pl.ds(base, B_PER_W)], n_v)

        lane = lax.iota(jnp.int32, L)
        total = jnp.zeros((L,), jnp.float32)
        for c in range(NCHUNK):
            off = c * CHUNK
            d1 = pltpu.async_copy(ent_hbm.at[h_v.at[pl.ds(off, CHUNK)]], he_v, sem)
            d2 = pltpu.async_copy(rel_hbm.at[r_v.at[pl.ds(off, CHUNK)]], re_v, sem)
            d3 = pltpu.async_copy(ent_hbm.at[t_v.at[pl.ds(off, CHUNK)]], te_v, sem)
            d4 = pltpu.async_copy(ent_hbm.at[n_v.at[pl.ds(off, CHUNK)]], ne_v, sem)
            d1.wait()
            d2.wait()
            d3.wait()
            d4.wait()

            def group_body(g, tot):
                rows = g * L + lane

                def col_body(col, acc):
                    cols = jnp.full((L,), col, jnp.int32)
                    sh = plsc.load_gather(he_v, [rows, cols])
                    sr = plsc.load_gather(re_v, [rows, cols])
                    st = plsc.load_gather(te_v, [rows, cols])
                    sn = plsc.load_gather(ne_v, [rows, cols])
                    s = sh + sr
                    return acc + (jnp.abs(s - st) - jnp.abs(s - sn))

                acc = lax.fori_loop(0, DIM, col_body, jnp.zeros((L,), jnp.float32))
                return tot + jnp.maximum(acc + 1.0, 0.0)

            total = lax.fori_loop(0, GROUPS, group_body, total)

        o_v[...] = total
        pltpu.sync_copy(o_v, out_hbm.at[pl.ds(wid * L, L)])

    return sc_kernel


_sc_kernel = _make_sc_kernel()


def _tc_reduce(partials):
    def body(x_ref, o_ref):
        o_ref[...] = jnp.sum(x_ref[...], keepdims=True).reshape(1, 1) * (1.0 / BATCH)

    return pl.pallas_call(
        body,
        out_shape=jax.ShapeDtypeStruct((1, 1), jnp.float32),
    )(partials)


def kernel(h, r, t, neg_t, entity_emb, relation_emb):
    h = h.astype(jnp.int32)
    r = r.astype(jnp.int32)
    t = t.astype(jnp.int32)
    neg_t = neg_t.astype(jnp.int32)
    ent_pad = jnp.pad(entity_emb, ((0, 0), (0, DIM)))
    rel_pad = jnp.pad(relation_emb, ((0, 0), (0, DIM)))
    partials = _sc_kernel(h, r, t, neg_t, ent_pad, rel_pad)
    return _tc_reduce(partials.reshape(NW, L)).reshape(())


# speedup vs baseline: 3.3457x; 3.0642x over previous
import functools

import jax
import jax.numpy as jnp
from jax import lax
from jax.experimental import pallas as pl
from jax.experimental.pallas import tpu as pltpu
from jax.experimental.pallas import tpu_sc as plsc

BATCH = 16384
DIM = 64
N_ENT = 1000000
NC = 2
NS = 16
L = 16
NW = NC * NS
B_PER_W = BATCH // NW
CHUNK = 64
NCHUNK = B_PER_W // CHUNK
GROUPS = CHUNK // L

PACK_BN = 16384
PACK_GRID = 31
N_PACKED = PACK_BN * PACK_GRID
TAIL_START = (N_ENT // PACK_BN) * PACK_BN


def _pack_table(xt):
    def body(x0_ref, x1_ref, o_ref):
        xcat = jnp.concatenate([x0_ref[...], x1_ref[...]], axis=0)
        for k in range(PACK_BN // 128):
            o_ref[pl.ds(k * 128, 128), :] = jnp.transpose(
                xcat[:, k * 128:(k + 1) * 128])

    return pl.pallas_call(
        body,
        grid=(PACK_GRID,),
        in_specs=[
            pl.BlockSpec((DIM, PACK_BN), lambda i: (0, i)),
            pl.BlockSpec((DIM, PACK_BN),
                         lambda i: (0, jnp.minimum(PACK_GRID + i,
                                                   N_ENT // PACK_BN - 1))),
        ],
        out_specs=pl.BlockSpec((PACK_BN, 2 * DIM), lambda i: (i, 0)),
        out_shape=jax.ShapeDtypeStruct((N_PACKED, 2 * DIM), jnp.float32),
    )(xt, xt)


def _make_sc_kernel():
    mesh = plsc.VectorSubcoreMesh(core_axis_name="c", subcore_axis_name="s")

    @functools.partial(
        pl.kernel,
        out_type=jax.ShapeDtypeStruct((NW * L,), jnp.float32),
        mesh=mesh,
        compiler_params=pltpu.CompilerParams(needs_layout_passes=False),
        scratch_types=[
            pltpu.VMEM((7, 1, B_PER_W), jnp.int32),
            pltpu.VMEM((CHUNK, 2 * DIM), jnp.float32),
            pltpu.VMEM((CHUNK, 2 * DIM), jnp.float32),
            pltpu.VMEM((CHUNK, 2 * DIM), jnp.float32),
            pltpu.VMEM((CHUNK, 2 * DIM), jnp.float32),
            pltpu.VMEM((CHUNK, 2 * DIM), jnp.float32),
            pltpu.VMEM((CHUNK, 2 * DIM), jnp.float32),
            pltpu.VMEM((CHUNK, 2 * DIM), jnp.float32),
            pltpu.VMEM((CHUNK, 2 * DIM), jnp.float32),
            pltpu.VMEM((L,), jnp.float32),
            pltpu.SemaphoreType.DMA,
            pltpu.SemaphoreType.DMA,
        ],
    )
    def sc_kernel(idx_hbm, ent_hbm, rel_hbm, out_hbm,
                  idx_v,
                  he_a, re_a, te_a, ne_a, he_b, re_b, te_b, ne_b,
                  o_v, sem_a, sem_b):
        wid = lax.axis_index("s") * NC + lax.axis_index("c")
        base = wid * B_PER_W
        pltpu.sync_copy(idx_hbm.at[:, :, pl.ds(base, B_PER_W)], idx_v)
        h_v, r_v, t_v, n_v = (idx_v.at[0, 0], idx_v.at[1, 0],
                              idx_v.at[2, 0], idx_v.at[3, 0])
        h2_v, t2_v, n2_v = idx_v.at[4, 0], idx_v.at[5, 0], idx_v.at[6, 0]

        lane = lax.iota(jnp.int32, L)
        total = jnp.zeros((L,), jnp.float32)
        bufsets = [(he_a, re_a, te_a, ne_a, sem_a),
                   (he_b, re_b, te_b, ne_b, sem_b)]

        def issue(c, bs):
            he_v, re_v, te_v, ne_v, sem = bs
            off = c * CHUNK
            return [
                pltpu.async_copy(ent_hbm.at[h2_v.at[pl.ds(off, CHUNK)]], he_v, sem),
                pltpu.async_copy(rel_hbm.at[r_v.at[pl.ds(off, CHUNK)]], re_v, sem),
                pltpu.async_copy(ent_hbm.at[t2_v.at[pl.ds(off, CHUNK)]], te_v, sem),
                pltpu.async_copy(ent_hbm.at[n2_v.at[pl.ds(off, CHUNK)]], ne_v, sem),
            ]

        descs = issue(0, bufsets[0])
        for c in range(NCHUNK):
            nxt = issue(c + 1, bufsets[(c + 1) % 2]) if c + 1 < NCHUNK else None
            for d in descs:
                d.wait()
            he_v, re_v, te_v, ne_v, _ = bufsets[c % 2]
            off = c * CHUNK

            def group_body(g, tot):
                rows = g * L + lane
                gsl = pl.ds(off + g * L, L)
                zero = jnp.zeros((L,), jnp.int32)
                dimv = jnp.full((L,), DIM, jnp.int32)
                hpar = jnp.where(h_v[gsl] >= N_PACKED, dimv, zero)
                tpar = jnp.where(t_v[gsl] >= N_PACKED, dimv, zero)
                npar = jnp.where(n_v[gsl] >= N_PACKED, dimv, zero)

                def col_body(col, acc):
                    cols = (col + lane) & (DIM - 1)
                    sh = plsc.load_gather(he_v, [rows, hpar + cols])
                    sr = plsc.load_gather(re_v, [rows, cols])
                    st = plsc.load_gather(te_v, [rows, tpar + cols])
                    sn = plsc.load_gather(ne_v, [rows, npar + cols])
                    s = sh + sr
                    return acc + (jnp.abs(s - st) - jnp.abs(s - sn))

                acc = lax.fori_loop(0, DIM, col_body, jnp.zeros((L,), jnp.float32),
                                    unroll=16)
                return tot + jnp.maximum(acc + 1.0, 0.0)

            total = lax.fori_loop(0, GROUPS, group_body, total)
            descs = nxt

        o_v[...] = total
        pltpu.sync_copy(o_v, out_hbm.at[pl.ds(wid * L, L)])

    return sc_kernel


_sc_kernel = _make_sc_kernel()


def _tc_reduce(partials):
    def body(x_ref, o_ref):
        o_ref[...] = jnp.sum(x_ref[...], keepdims=True).reshape(1, 1) * (1.0 / BATCH)

    return pl.pallas_call(
        body,
        out_shape=jax.ShapeDtypeStruct((1, 1), jnp.float32),
    )(partials)


def kernel(h, r, t, neg_t, entity_emb, relation_emb):
    h = h.astype(jnp.int32)
    r = r.astype(jnp.int32)
    t = t.astype(jnp.int32)
    neg_t = neg_t.astype(jnp.int32)
    packed = _pack_table(entity_emb.T)
    tail = entity_emb[TAIL_START:]
    packed = packed.at[TAIL_START - N_PACKED:N_ENT - N_PACKED, DIM:].set(tail)
    rel_pad = jnp.pad(relation_emb, ((0, 0), (0, DIM)))
    h2 = jnp.where(h >= N_PACKED, h - N_PACKED, h)
    t2 = jnp.where(t >= N_PACKED, t - N_PACKED, t)
    n2 = jnp.where(neg_t >= N_PACKED, neg_t - N_PACKED, neg_t)
    idx_all = jnp.stack([h, r, t, neg_t, h2, t2, n2]).reshape(7, 1, BATCH)
    partials = _sc_kernel(idx_all, packed, rel_pad)
    return _tc_reduce(partials.reshape(NW, L)).reshape(())
